# baseline (speedup 1.0000x reference)
import functools

import jax
import jax.numpy as jnp
from jax.experimental import pallas as pl
from jax.experimental.pallas import tpu as pltpu

CELLS = 16384
OUT_DIM = 32
FEATS = 64
BLOCK = 256
GRID = CELLS // BLOCK


def _body(idx_ref, x_ref, mask_ref, w_ref, out_ref):
    i = pl.program_id(0)

    @pl.when(i == 0)
    def _init():
        out_ref[...] = jnp.zeros_like(out_ref)

    w = w_ref[...].reshape(1, 1, FEATS)
    dense = jnp.sum(x_ref[...] * w, axis=-1)
    base = i * BLOCK
    out_ref[pl.ds(base, BLOCK), :] += dense

    def scatter(c, _):
        row = idx_ref[0, 0, c]
        out_ref[pl.ds(row, 1), :] += mask_ref[pl.ds(c, 1), :]
        return 0

    jax.lax.fori_loop(0, BLOCK, scatter, 0)


@jax.jit
def kernel(x, index, mask, w1, w2):
    w = jnp.concatenate([w1, w2], axis=-1)
    idx3 = index.astype(jnp.int32).reshape(GRID, 1, BLOCK)
    return pl.pallas_call(
        _body,
        grid=(GRID,),
        in_specs=[
            pl.BlockSpec((1, 1, BLOCK), lambda i: (i, 0, 0),
                         memory_space=pltpu.SMEM),
            pl.BlockSpec((BLOCK, OUT_DIM, FEATS), lambda i: (i, 0, 0)),
            pl.BlockSpec((BLOCK, OUT_DIM), lambda i: (i, 0)),
            pl.BlockSpec((1, FEATS), lambda i: (0, 0)),
        ],
        out_specs=pl.BlockSpec((CELLS, OUT_DIM), lambda i: (0, 0)),
        out_shape=jax.ShapeDtypeStruct((CELLS, OUT_DIM), jnp.float32),
        compiler_params=pltpu.CompilerParams(
            dimension_semantics=("arbitrary",),
        ),
    )(idx3, x, mask, w)
